# baseline (device time: 1808260 ns/iter reference)
import jax
import jax.numpy as jnp
from jax import lax
from jax.experimental import pallas as pl
from jax.experimental.pallas import tpu as pltpu

N_DEV = 4
M_PER = 2048
K = 8192
N_PER = 1024
HALF = M_PER // 2
R = 4
U = HALF // R
NPOS = 3 * R
NSLOT = 3


def _all_gather_bidir(x):

    def body(x_ref, gx_ref, slots, own_sem, copy_sems, send_sems, recv_sems):
        my = lax.axis_index("i")
        right = lax.rem(my + 1, N_DEV)
        left = lax.rem(my + N_DEV - 1, N_DEV)

        copies = []
        for r in range(R):
            for h in range(3):
                p = r * 3 + h
                rdmas = []
                for d in range(2):
                    tgt = right if d == 0 else left
                    base = 0 if d == 0 else HALF
                    if h == 0:
                        src = x_ref.at[pl.ds(base + r * U, U)]
                    else:
                        src = slots.at[d, (p - 1) % NSLOT]
                    rd = pltpu.make_async_remote_copy(
                        src_ref=src,
                        dst_ref=slots.at[d, p % NSLOT],
                        send_sem=send_sems.at[d, p],
                        recv_sem=recv_sems.at[d, p],
                        device_id=(tgt,),
                        device_id_type=pl.DeviceIdType.MESH,
                    )
                    rd.start()
                    rdmas.append(rd)
                for rd in rdmas:
                    rd.wait()
                for d in range(2):
                    if d == 0:
                        o = lax.rem(my + N_DEV - 1 - h, N_DEV)
                    else:
                        o = lax.rem(my + 1 + h, N_DEV)
                    row = o * M_PER + (0 if d == 0 else HALF) + r * U
                    c = pltpu.make_async_copy(
                        slots.at[d, p % NSLOT],
                        gx_ref.at[pl.ds(row, U)],
                        copy_sems.at[d, p],
                    )
                    c.start()
                    copies.append(c)

        own = pltpu.make_async_copy(
            x_ref, gx_ref.at[pl.ds(my * M_PER, M_PER)], own_sem
        )
        own.start()
        own.wait()
        for c in copies:
            c.wait()

    return pl.pallas_call(
        body,
        out_shape=jax.ShapeDtypeStruct((N_DEV * M_PER, K), jnp.bfloat16),
        in_specs=[pl.BlockSpec(memory_space=pl.ANY)],
        out_specs=pl.BlockSpec(memory_space=pl.ANY),
        scratch_shapes=[
            pltpu.VMEM((2, NSLOT, U, K), jnp.bfloat16),
            pltpu.SemaphoreType.DMA,
            pltpu.SemaphoreType.DMA((2, NPOS)),
            pltpu.SemaphoreType.DMA((2, NPOS)),
            pltpu.SemaphoreType.DMA((2, NPOS)),
        ],
    )(x)


def _gemm_gelu(gx, w):
    M = N_DEV * M_PER
    bm, bk, bn = 512, 2048, N_PER
    n_k = K // bk

    def body(x_ref, w_ref, o_ref, acc_ref):
        k = pl.program_id(1)

        @pl.when(k == 0)
        def _():
            acc_ref[...] = jnp.zeros_like(acc_ref)

        acc_ref[...] += jnp.dot(
            x_ref[...], w_ref[...], preferred_element_type=jnp.float32
        )

        @pl.when(k == n_k - 1)
        def _():
            o_ref[...] = jax.nn.gelu(acc_ref[...], approximate=True)

    return pl.pallas_call(
        body,
        grid=(M // bm, n_k),
        in_specs=[
            pl.BlockSpec((bm, bk), lambda m, k: (m, k)),
            pl.BlockSpec((bk, bn), lambda m, k: (k, 0)),
        ],
        out_specs=pl.BlockSpec((bm, bn), lambda m, k: (m, 0)),
        out_shape=jax.ShapeDtypeStruct((M, N_PER), jnp.float32),
        scratch_shapes=[pltpu.VMEM((bm, bn), jnp.float32)],
    )(gx, w)


def kernel(x, w_mat):
    x = x.astype(jnp.bfloat16)
    w_mat = w_mat.astype(jnp.bfloat16)
    gx = _all_gather_bidir(x)
    return _gemm_gelu(gx, w_mat)


# device time: 823885 ns/iter; 2.1948x vs baseline; 2.1948x over previous
import jax
import jax.numpy as jnp
from jax import lax
from jax.experimental import pallas as pl
from jax.experimental.pallas import tpu as pltpu

N_DEV = 4
M_PER = 2048
K = 8192
N_PER = 1024
HALF = M_PER // 2
R = 4
U = HALF // R
NPOS = 3 * R
NSLOT = 3


def _all_gather_bidir(x):

    def body(x_ref, gx_ref, slots, bounce, own_sem, copy_sems, send_sems,
             recv_sems):
        my = lax.axis_index("i")
        right = lax.rem(my + 1, N_DEV)
        left = lax.rem(my + N_DEV - 1, N_DEV)

        copies = []
        for r in range(R):
            for h in range(3):
                p = r * 3 + h
                rdmas = []
                for d in range(2):
                    tgt = right if d == 0 else left
                    base = 0 if d == 0 else HALF
                    if h == 0:
                        src = x_ref.at[pl.ds(base + r * U, U)]
                    else:
                        src = slots.at[d, (p - 1) % NSLOT]
                    rd = pltpu.make_async_remote_copy(
                        src_ref=src,
                        dst_ref=slots.at[d, p % NSLOT],
                        send_sem=send_sems.at[d, p],
                        recv_sem=recv_sems.at[d, p],
                        device_id=(tgt,),
                        device_id_type=pl.DeviceIdType.MESH,
                    )
                    rd.start()
                    rdmas.append(rd)
                for rd in rdmas:
                    rd.wait()
                for d in range(2):
                    if d == 0:
                        o = lax.rem(my + N_DEV - 1 - h, N_DEV)
                    else:
                        o = lax.rem(my + 1 + h, N_DEV)
                    row = o * M_PER + (0 if d == 0 else HALF) + r * U
                    c = pltpu.make_async_copy(
                        slots.at[d, p % NSLOT],
                        gx_ref.at[pl.ds(row, U)],
                        copy_sems.at[d, p],
                    )
                    c.start()
                    copies.append(c)

        for j in range(M_PER // U):
            b = pltpu.make_async_copy(
                x_ref.at[pl.ds(j * U, U)], bounce.at[j % 2], own_sem
            )
            b.start()
            b.wait()
            b2 = pltpu.make_async_copy(
                bounce.at[j % 2],
                gx_ref.at[pl.ds(my * M_PER + j * U, U)],
                own_sem,
            )
            b2.start()
            b2.wait()
        for c in copies:
            c.wait()

    return pl.pallas_call(
        body,
        out_shape=jax.ShapeDtypeStruct((N_DEV * M_PER, K), jnp.bfloat16),
        in_specs=[pl.BlockSpec(memory_space=pl.ANY)],
        out_specs=pl.BlockSpec(memory_space=pl.ANY),
        scratch_shapes=[
            pltpu.VMEM((2, NSLOT, U, K), jnp.bfloat16),
            pltpu.VMEM((2, U, K), jnp.bfloat16),
            pltpu.SemaphoreType.DMA,
            pltpu.SemaphoreType.DMA((2, NPOS)),
            pltpu.SemaphoreType.DMA((2, NPOS)),
            pltpu.SemaphoreType.DMA((2, NPOS)),
        ],
    )(x)


def _gemm_gelu(gx, w):
    M = N_DEV * M_PER
    bm, bk, bn = 512, 2048, N_PER
    n_k = K // bk

    def body(x_ref, w_ref, o_ref, acc_ref):
        k = pl.program_id(1)

        @pl.when(k == 0)
        def _():
            acc_ref[...] = jnp.zeros_like(acc_ref)

        acc_ref[...] += jnp.dot(
            x_ref[...], w_ref[...], preferred_element_type=jnp.float32
        )

        @pl.when(k == n_k - 1)
        def _():
            o_ref[...] = jax.nn.gelu(acc_ref[...], approximate=True)

    return pl.pallas_call(
        body,
        grid=(M // bm, n_k),
        in_specs=[
            pl.BlockSpec((bm, bk), lambda m, k: (m, k)),
            pl.BlockSpec((bk, bn), lambda m, k: (k, 0)),
        ],
        out_specs=pl.BlockSpec((bm, bn), lambda m, k: (m, 0)),
        out_shape=jax.ShapeDtypeStruct((M, N_PER), jnp.float32),
        scratch_shapes=[pltpu.VMEM((bm, bn), jnp.float32)],
    )(gx, w)


def kernel(x, w_mat):
    x = x.astype(jnp.bfloat16)
    w_mat = w_mat.astype(jnp.bfloat16)
    gx = _all_gather_bidir(x)
    return _gemm_gelu(gx, w_mat)


# device time: 823659 ns/iter; 2.1954x vs baseline; 1.0003x over previous
import jax
import jax.numpy as jnp
from jax import lax
from jax.experimental import pallas as pl
from jax.experimental.pallas import tpu as pltpu

N_DEV = 4
M_PER = 2048
K = 8192
N_PER = 1024
HALF = M_PER // 2
R = 4
U = HALF // R
NPOS = 3 * R
NSLOT = 3

RF = 2
UF = HALF // RF
NPOSF = 3 * RF
NSLOTF = 2


def _all_gather_bidir(x):

    def body(x_ref, gx_ref, slots, bounce, own_sem, copy_sems, send_sems,
             recv_sems):
        my = lax.axis_index("i")
        right = lax.rem(my + 1, N_DEV)
        left = lax.rem(my + N_DEV - 1, N_DEV)

        copies = []
        for r in range(R):
            for h in range(3):
                p = r * 3 + h
                rdmas = []
                for d in range(2):
                    tgt = right if d == 0 else left
                    base = 0 if d == 0 else HALF
                    if h == 0:
                        src = x_ref.at[pl.ds(base + r * U, U)]
                    else:
                        src = slots.at[d, (p - 1) % NSLOT]
                    rd = pltpu.make_async_remote_copy(
                        src_ref=src,
                        dst_ref=slots.at[d, p % NSLOT],
                        send_sem=send_sems.at[d, p],
                        recv_sem=recv_sems.at[d, p],
                        device_id=(tgt,),
                        device_id_type=pl.DeviceIdType.MESH,
                    )
                    rd.start()
                    rdmas.append(rd)
                for rd in rdmas:
                    rd.wait()
                for d in range(2):
                    if d == 0:
                        o = lax.rem(my + N_DEV - 1 - h, N_DEV)
                    else:
                        o = lax.rem(my + 1 + h, N_DEV)
                    row = o * M_PER + (0 if d == 0 else HALF) + r * U
                    c = pltpu.make_async_copy(
                        slots.at[d, p % NSLOT],
                        gx_ref.at[pl.ds(row, U)],
                        copy_sems.at[d, p],
                    )
                    c.start()
                    copies.append(c)

        for j in range(M_PER // U):
            b = pltpu.make_async_copy(
                x_ref.at[pl.ds(j * U, U)], bounce.at[j % 2], own_sem
            )
            b.start()
            b.wait()
            b2 = pltpu.make_async_copy(
                bounce.at[j % 2],
                gx_ref.at[pl.ds(my * M_PER + j * U, U)],
                own_sem,
            )
            b2.start()
            b2.wait()
        for c in copies:
            c.wait()

    return pl.pallas_call(
        body,
        out_shape=jax.ShapeDtypeStruct((N_DEV * M_PER, K), jnp.bfloat16),
        in_specs=[pl.BlockSpec(memory_space=pl.ANY)],
        out_specs=pl.BlockSpec(memory_space=pl.ANY),
        scratch_shapes=[
            pltpu.VMEM((2, NSLOT, U, K), jnp.bfloat16),
            pltpu.VMEM((2, U, K), jnp.bfloat16),
            pltpu.SemaphoreType.DMA,
            pltpu.SemaphoreType.DMA((2, NPOS)),
            pltpu.SemaphoreType.DMA((2, NPOS)),
            pltpu.SemaphoreType.DMA((2, NPOS)),
        ],
    )(x)


def _gemm_gelu(gx, w):
    M = N_DEV * M_PER
    bm, bk, bn = 512, 2048, N_PER
    n_k = K // bk

    def body(x_ref, w_ref, o_ref, acc_ref):
        k = pl.program_id(1)

        @pl.when(k == 0)
        def _():
            acc_ref[...] = jnp.zeros_like(acc_ref)

        acc_ref[...] += jnp.dot(
            x_ref[...], w_ref[...], preferred_element_type=jnp.float32
        )

        @pl.when(k == n_k - 1)
        def _():
            o_ref[...] = jax.nn.gelu(acc_ref[...], approximate=True)

    return pl.pallas_call(
        body,
        grid=(M // bm, n_k),
        in_specs=[
            pl.BlockSpec((bm, bk), lambda m, k: (m, k)),
            pl.BlockSpec((bk, bn), lambda m, k: (k, 0)),
        ],
        out_specs=pl.BlockSpec((bm, bn), lambda m, k: (m, 0)),
        out_shape=jax.ShapeDtypeStruct((M, N_PER), jnp.float32),
        scratch_shapes=[pltpu.VMEM((bm, bn), jnp.float32)],
    )(gx, w)


def _own_gemm(x, w):
    bm, bk, bn = 512, 2048, N_PER
    n_k = K // bk

    def body(x_ref, w_ref, o_ref, acc_ref):
        k = pl.program_id(1)

        @pl.when(k == 0)
        def _():
            acc_ref[...] = jnp.zeros_like(acc_ref)

        acc_ref[...] += jnp.dot(
            x_ref[...], w_ref[...], preferred_element_type=jnp.float32
        )

        @pl.when(k == n_k - 1)
        def _():
            o_ref[...] = jax.nn.gelu(acc_ref[...], approximate=True)

    return pl.pallas_call(
        body,
        grid=(M_PER // bm, n_k),
        in_specs=[
            pl.BlockSpec((bm, bk), lambda m, k: (m, k)),
            pl.BlockSpec((bk, bn), lambda m, k: (k, 0)),
        ],
        out_specs=pl.BlockSpec((bm, bn), lambda m, k: (m, 0)),
        out_shape=jax.ShapeDtypeStruct((M_PER, N_PER), jnp.float32),
        scratch_shapes=[pltpu.VMEM((bm, bn), jnp.float32)],
    )(x, w)


def _ring_fused(x, w, y_own):
    R, U, NPOS, NSLOT = RF, UF, NPOSF, NSLOTF

    def compute_unit(slots, w_ref, ystage, out_ref, ycopy_sems, my, p):
        r, h = p // 3, p % 3
        started = []
        for d in range(2):
            o = (
                lax.rem(my + N_DEV - 1 - h, N_DEV)
                if d == 0
                else lax.rem(my + 1 + h, N_DEV)
            )
            row = o * M_PER + (0 if d == 0 else HALF) + r * U
            ystage[d, :, :] = jax.nn.gelu(
                jnp.dot(
                    slots[d, p % NSLOT],
                    w_ref[...],
                    preferred_element_type=jnp.float32,
                ),
                approximate=True,
            )
            c = pltpu.make_async_copy(
                ystage.at[d],
                out_ref.at[pl.ds(row, U)],
                ycopy_sems.at[d, p],
            )
            c.start()
            started.append(c)
        return started

    def body(x_ref, w_ref, yown_ref, out_ref, slots, ystage, own_sem,
             ycopy_sems, send_sems, recv_sems):
        my = lax.axis_index("i")
        right = lax.rem(my + 1, N_DEV)
        left = lax.rem(my + N_DEV - 1, N_DEV)

        ycopies = {}
        for r in range(R):
            for h in range(3):
                p = r * 3 + h
                rdmas = []
                for d in range(2):
                    tgt = right if d == 0 else left
                    base = 0 if d == 0 else HALF
                    if h == 0:
                        src = x_ref.at[pl.ds(base + r * U, U)]
                    else:
                        src = slots.at[d, (p - 1) % NSLOT]
                    rd = pltpu.make_async_remote_copy(
                        src_ref=src,
                        dst_ref=slots.at[d, p % NSLOT],
                        send_sem=send_sems.at[d, p],
                        recv_sem=recv_sems.at[d, p],
                        device_id=(tgt,),
                        device_id_type=pl.DeviceIdType.MESH,
                    )
                    rd.start()
                    rdmas.append(rd)
                if p > 0:
                    if p >= 2:
                        for c in ycopies[p - 2]:
                            c.wait()
                    ycopies[p - 1] = compute_unit(
                        slots, w_ref, ystage, out_ref, ycopy_sems, my, p - 1
                    )
                for rd in rdmas:
                    rd.wait()

        for c in ycopies[NPOS - 2]:
            c.wait()
        ycopies[NPOS - 1] = compute_unit(
            slots, w_ref, ystage, out_ref, ycopy_sems, my, NPOS - 1
        )
        for c in ycopies[NPOS - 1]:
            c.wait()

        for j in range(M_PER // U):
            b = pltpu.make_async_copy(
                yown_ref.at[pl.ds(j * U, U)], ystage.at[j % 2], own_sem
            )
            b.start()
            b.wait()
            b2 = pltpu.make_async_copy(
                ystage.at[j % 2],
                out_ref.at[pl.ds(my * M_PER + j * U, U)],
                own_sem,
            )
            b2.start()
            b2.wait()

    return pl.pallas_call(
        body,
        out_shape=jax.ShapeDtypeStruct((N_DEV * M_PER, N_PER), jnp.float32),
        in_specs=[
            pl.BlockSpec(memory_space=pl.ANY),
            pl.BlockSpec(memory_space=pltpu.VMEM),
            pl.BlockSpec(memory_space=pl.ANY),
        ],
        out_specs=pl.BlockSpec(memory_space=pl.ANY),
        scratch_shapes=[
            pltpu.VMEM((2, NSLOT, U, K), jnp.bfloat16),
            pltpu.VMEM((2, U, N_PER), jnp.float32),
            pltpu.SemaphoreType.DMA,
            pltpu.SemaphoreType.DMA((2, NPOS)),
            pltpu.SemaphoreType.DMA((2, NPOS)),
            pltpu.SemaphoreType.DMA((2, NPOS)),
        ],
        compiler_params=pltpu.CompilerParams(
            vmem_limit_bytes=60 * 1024 * 1024
        ),
    )(x, w, y_own)


def kernel(x, w_mat):
    x = x.astype(jnp.bfloat16)
    w_mat = w_mat.astype(jnp.bfloat16)
    gx = _all_gather_bidir(x)
    return _gemm_gelu(gx, w_mat)
